# baseline (speedup 1.0000x reference)
import functools

import jax
import jax.numpy as jnp
from jax import lax
from jax.experimental import pallas as pl
from jax.experimental.pallas import tpu as pltpu
from jax.experimental.pallas import tpu_sc as plsc

SCALE = 30.0
NC = 2
NS = 16
L = 16
NW = NC * NS


def _fast_log(x):
    bits = lax.bitcast_convert_type(x, jnp.int32)
    e = (bits >> 23) - 127
    m = lax.bitcast_convert_type(
        (bits & 0x007FFFFF) | 0x3F800000, jnp.float32)
    big = m > 1.4142135623730951
    m = jnp.where(big, m * 0.5, m)
    e = jnp.where(big, e + 1, e)
    z = (m - 1.0) / (m + 1.0)
    z2 = z * z
    p = 2.0 * z * (1.0 + z2 * (1.0 / 3.0 + z2 * (0.2 + z2 * (1.0 / 7.0))))
    return e.astype(jnp.float32) * 0.6931471805599453 + p


def _make_sc_kernel(B, C):
    rows_w = B // NW
    groups = rows_w // L
    inv_b = 1.0 / B

    mesh = plsc.VectorSubcoreMesh(core_axis_name="c", subcore_axis_name="s")

    @functools.partial(
        pl.kernel,
        out_type=jax.ShapeDtypeStruct((NW, L), jnp.float32),
        mesh=mesh,
        compiler_params=pltpu.CompilerParams(needs_layout_passes=False),
        scratch_types=[
            pltpu.VMEM((rows_w, C), jnp.float32),
            pltpu.VMEM((rows_w,), jnp.int32),
            pltpu.VMEM((C,), jnp.float32),
            pltpu.VMEM((L,), jnp.float32),
        ],
    )
    def body(logits_hbm, targets_hbm, mlist_hbm, out_hbm,
             logits_v, targets_v, mlist_v, stage_v):
        cid = lax.axis_index("c")
        sid = lax.axis_index("s")
        wid = cid * NS + sid

        pltpu.sync_copy(logits_hbm.at[pl.ds(wid * rows_w, rows_w), :], logits_v)
        pltpu.sync_copy(targets_hbm.at[pl.ds(wid * rows_w, rows_w)], targets_v)
        pltpu.sync_copy(mlist_hbm, mlist_v)

        lane = lax.iota(jnp.int32, 16)
        zeros16 = lane * 0

        U = 10

        def group_body(g, acc):
            rows = g * L + lane
            t = plsc.load_gather(targets_v, [rows])
            mg = plsc.load_gather(mlist_v, [t])
            st = plsc.load_gather(logits_v, [rows, t])

            def max_body(jj, ms):
                j = jj * U
                return tuple(
                    jnp.maximum(ms[u],
                                plsc.load_gather(logits_v,
                                                 [rows, zeros16 + (j + u)]))
                    for u in range(U))

            neg = jnp.full((L,), -jnp.inf, jnp.float32)
            ms = list(lax.fori_loop(0, C // U, max_body, (neg,) * U))
            while len(ms) > 1:
                ms = [jnp.maximum(ms[i], ms[i + 1]) if i + 1 < len(ms)
                      else ms[i] for i in range(0, len(ms), 2)]
            mx = ms[0]

            def sum_body(jj, ss):
                j = jj * U
                out = []
                for u in range(U):
                    v = plsc.load_gather(logits_v, [rows, zeros16 + (j + u)])
                    v = v - jnp.where(t == j + u, mg, 0.0)
                    out.append(ss[u] + jnp.exp((v - mx) * SCALE))
                return tuple(out)

            zero = jnp.zeros((L,), jnp.float32)
            ss = list(lax.fori_loop(0, C // U, sum_body, (zero,) * U))
            while len(ss) > 1:
                ss = [ss[i] + ss[i + 1] if i + 1 < len(ss)
                      else ss[i] for i in range(0, len(ss), 2)]
            s = ss[0]
            nll = SCALE * (mx - st + mg) + _fast_log(s)
            return acc + nll

        acc = lax.fori_loop(0, groups, group_body,
                            jnp.zeros((L,), jnp.float32))

        stage_v[...] = acc * inv_b
        pltpu.sync_copy(stage_v, out_hbm.at[wid])

    return body


def _finish_body(parts_ref, out_ref):
    out_ref[0, 0] = jnp.sum(parts_ref[...])


def kernel(logits, targets, m_list):
    B, C = logits.shape
    sc = _make_sc_kernel(B, C)
    parts = sc(logits, targets, m_list)
    total = pl.pallas_call(
        _finish_body,
        out_shape=jax.ShapeDtypeStruct((1, 1), jnp.float32),
        out_specs=pl.BlockSpec(memory_space=pltpu.SMEM),
    )(parts)
    return total.reshape(())


# speedup vs baseline: 1.3827x; 1.3827x over previous
import functools

import jax
import jax.numpy as jnp
from jax import lax
from jax.experimental import pallas as pl
from jax.experimental.pallas import tpu as pltpu
from jax.experimental.pallas import tpu_sc as plsc

SCALE = 30.0
NC = 2
NS = 16
L = 16
NW = NC * NS


def _fast_log(x):
    bits = lax.bitcast_convert_type(x, jnp.int32)
    e = (bits >> 23) - 127
    m = lax.bitcast_convert_type(
        (bits & 0x007FFFFF) | 0x3F800000, jnp.float32)
    big = m > 1.4142135623730951
    m = jnp.where(big, m * 0.5, m)
    e = jnp.where(big, e + 1, e)
    z = (m - 1.0) / (m + 1.0)
    z2 = z * z
    p = 2.0 * z * (1.0 + z2 * (1.0 / 3.0 + z2 * (0.2 + z2 * (1.0 / 7.0))))
    return e.astype(jnp.float32) * 0.6931471805599453 + p


def _make_sc_kernel(B, C):
    rows_w = B // NW
    words_w = rows_w * C
    prows_w = words_w // 128
    groups = rows_w // L
    inv_b = 1.0 / B

    mesh = plsc.VectorSubcoreMesh(core_axis_name="c", subcore_axis_name="s")

    @functools.partial(
        pl.kernel,
        out_type=jax.ShapeDtypeStruct((NW, L), jnp.float32),
        mesh=mesh,
        compiler_params=pltpu.CompilerParams(needs_layout_passes=False),
        scratch_types=[
            pltpu.VMEM((prows_w, 128), jnp.float32),
            pltpu.VMEM((rows_w,), jnp.int32),
            pltpu.VMEM((C,), jnp.float32),
            pltpu.VMEM((L,), jnp.float32),
        ],
    )
    def body(logits_hbm, targets_hbm, mlist_hbm, out_hbm,
             logits_v, targets_v, mlist_v, stage_v):
        cid = lax.axis_index("c")
        sid = lax.axis_index("s")
        wid = cid * NS + sid

        pltpu.sync_copy(logits_hbm.at[pl.ds(wid * prows_w, prows_w), :],
                        logits_v)
        pltpu.sync_copy(targets_hbm.at[pl.ds(wid * rows_w, rows_w)], targets_v)
        pltpu.sync_copy(mlist_hbm, mlist_v)

        lane = lax.iota(jnp.int32, 16)

        U = 10

        def gather_flat(o):
            return plsc.load_gather(logits_v, [o >> 7, o & 127])

        def group_body(g, acc):
            rows = g * L + lane
            ob = rows * C
            t = plsc.load_gather(targets_v, [rows])
            mg = plsc.load_gather(mlist_v, [t])
            st = gather_flat(ob + t)

            def max_body(jj, ms):
                j = jj * U
                return tuple(
                    jnp.maximum(ms[u], gather_flat(ob + (j + u)))
                    for u in range(U))

            neg = jnp.full((L,), -jnp.inf, jnp.float32)
            ms = list(lax.fori_loop(0, C // U, max_body, (neg,) * U))
            while len(ms) > 1:
                ms = [jnp.maximum(ms[i], ms[i + 1]) if i + 1 < len(ms)
                      else ms[i] for i in range(0, len(ms), 2)]
            mx = ms[0]

            def sum_body(jj, ss):
                j = jj * U
                out = []
                for u in range(U):
                    v = gather_flat(ob + (j + u))
                    v = v - jnp.where(t == j + u, mg, 0.0)
                    out.append(ss[u] + jnp.exp((v - mx) * SCALE))
                return tuple(out)

            zero = jnp.zeros((L,), jnp.float32)
            ss = list(lax.fori_loop(0, C // U, sum_body, (zero,) * U))
            while len(ss) > 1:
                ss = [ss[i] + ss[i + 1] if i + 1 < len(ss)
                      else ss[i] for i in range(0, len(ss), 2)]
            s = ss[0]
            nll = SCALE * (mx - st + mg) + _fast_log(s)
            return acc + nll

        acc = lax.fori_loop(0, groups, group_body,
                            jnp.zeros((L,), jnp.float32))

        stage_v[...] = acc * inv_b
        pltpu.sync_copy(stage_v, out_hbm.at[wid])

    return body


def _finish_body(parts_ref, out_ref):
    out_ref[0, 0] = jnp.sum(parts_ref[...])


def kernel(logits, targets, m_list):
    B, C = logits.shape
    sc = _make_sc_kernel(B, C)
    parts = sc(logits.reshape(B * C // 128, 128), targets, m_list)
    total = pl.pallas_call(
        _finish_body,
        out_shape=jax.ShapeDtypeStruct((1, 1), jnp.float32),
        out_specs=pl.BlockSpec(memory_space=pltpu.SMEM),
    )(parts)
    return total.reshape(())
